# Initial kernel scaffold; baseline (speedup 1.0000x reference)
import jax
import jax.numpy as jnp
from jax.experimental import pallas as pl


def kernel(x, edge_index, edge_attr, batch, params):
    raise NotImplementedError("write your pallas kernel here")


# speedup vs baseline: 3.5235x; 3.5235x over previous
import functools

import jax
import jax.numpy as jnp
from jax import lax
from jax.experimental import pallas as pl
from jax.experimental.pallas import tpu as pltpu
from jax.experimental.pallas import tpu_sc as plsc

N = 10000
E = 320000
D = 256
DQ = 64
NC = 2
NS = 16
EK = 80
NP = 10112

_MESH = dict(core_axis_name="c", subcore_axis_name="s")


def _sc_degree(src):
    ones = jnp.ones((EK,), jnp.float32)
    zer = jnp.zeros((632,), jnp.float32)

    @functools.partial(
        pl.kernel,
        out_type=[jax.ShapeDtypeStruct((NP,), jnp.float32),
                  jax.ShapeDtypeStruct((NP,), jnp.float32)],
        mesh=plsc.VectorSubcoreMesh(**_MESH),
        scratch_types=[
            pltpu.VMEM((EK,), jnp.int32),
            pltpu.VMEM((EK,), jnp.float32),
            pltpu.VMEM((632,), jnp.float32),
            pltpu.VMEM_SHARED((NP,), jnp.float32),
        ],
    )
    def deg_kernel(src_hbm, ones_hbm, zer_hbm, out0_hbm, out1_hbm,
                   idx_v, ones_v, stage_v, acc):
        c = lax.axis_index("c")
        s = lax.axis_index("s")

        pltpu.sync_copy(zer_hbm, stage_v)

        @pl.when(s < NS - 1)
        def _():
            pltpu.sync_copy(stage_v, acc.at[pl.ds(s * 632, 632)])

        @pl.when(s == NS - 1)
        def _():
            pltpu.sync_copy(stage_v.at[pl.ds(0, 520)],
                            acc.at[pl.ds((NS - 1) * 632, 520)])

        pltpu.sync_copy(ones_hbm, ones_v)
        plsc.subcore_barrier()

        per_w = E // (NC * NS)
        base = (c * NS + s) * per_w

        def step(k, carry):
            pltpu.sync_copy(src_hbm.at[pl.ds(base + k * EK, EK)], idx_v)
            pltpu.sync_copy(ones_v, acc.at[idx_v], add=True)
            return carry

        lax.fori_loop(0, per_w // EK, step, 0)
        plsc.subcore_barrier()

        for ci, out_hbm in enumerate((out0_hbm, out1_hbm)):
            @pl.when((c == ci) & (s < NS - 1))
            def _(out_hbm=out_hbm):
                pltpu.sync_copy(acc.at[pl.ds(s * 632, 632)], stage_v)
                pltpu.sync_copy(stage_v, out_hbm.at[pl.ds(s * 632, 632)])

            @pl.when((c == ci) & (s == NS - 1))
            def _(out_hbm=out_hbm):
                pltpu.sync_copy(acc.at[pl.ds((NS - 1) * 632, 520)],
                                stage_v.at[pl.ds(0, 520)])
                pltpu.sync_copy(stage_v.at[pl.ds(0, 520)],
                                out_hbm.at[pl.ds((NS - 1) * 632, 520)])

    return deg_kernel(src, ones, zer)


def _sc_aggregate(hs_tab, src4, dst):
    zrows = jnp.zeros((128, DQ), jnp.float32)

    @functools.partial(
        pl.kernel,
        out_type=[jax.ShapeDtypeStruct((N, DQ), jnp.float32)] * 4,
        mesh=plsc.VectorSubcoreMesh(**_MESH),
        scratch_types=[
            pltpu.VMEM((EK,), jnp.int32),
            pltpu.VMEM((EK,), jnp.int32),
            pltpu.VMEM((EK, DQ), jnp.float32),
            pltpu.VMEM((128, DQ), jnp.float32),
            pltpu.VMEM_SHARED((N, DQ), jnp.float32),
            pltpu.SemaphoreType.DMA,
        ],
        compiler_params=pltpu.CompilerParams(use_tc_tiling_on_sc=False),
    )
    def agg_kernel(tab_hbm, src_hbm, dst_hbm, zr_hbm,
                   o0_hbm, o1_hbm, o2_hbm, o3_hbm,
                   idx_s, idx_d, rows, zbuf, acc, sem):
        c = lax.axis_index("c")
        s = lax.axis_index("s")

        pltpu.sync_copy(zr_hbm, zbuf)

        def zero_own_range():
            @pl.when(s < NS - 1)
            def _():
                def zs(j, carry):
                    pltpu.sync_copy(zbuf, acc.at[pl.ds(s * 640 + j * 128, 128)])
                    return carry
                lax.fori_loop(0, 5, zs, 0)

            @pl.when(s == NS - 1)
            def _():
                def zs(j, carry):
                    pltpu.sync_copy(zbuf.at[pl.ds(0, 80)],
                                    acc.at[pl.ds(9600 + j * 80, 80)])
                    return carry
                lax.fori_loop(0, 5, zs, 0)

        def dump(out_hbm):
            @pl.when(s < NS - 1)
            def _():
                def dsx(j, carry):
                    r = s * 640 + j * 128
                    pltpu.sync_copy(acc.at[pl.ds(r, 128)], rows_stage)
                    pltpu.sync_copy(rows_stage, out_hbm.at[pl.ds(r, 128)])
                    return carry
                lax.fori_loop(0, 5, dsx, 0)

            @pl.when(s == NS - 1)
            def _():
                def dsx(j, carry):
                    r = 9600 + j * 80
                    pltpu.sync_copy(acc.at[pl.ds(r, 80)],
                                    rows_stage.at[pl.ds(0, 80)])
                    pltpu.sync_copy(rows_stage.at[pl.ds(0, 80)],
                                    out_hbm.at[pl.ds(r, 80)])
                    return carry
                lax.fori_loop(0, 5, dsx, 0)

        rows_stage = zbuf

        per_s = E // NS
        ebase = s * per_s

        def edges(qi):
            def step(k, carry):
                b = ebase + k * EK
                pltpu.sync_copy(src_hbm.at[pl.ds(qi * E + b, EK)], idx_s)
                pltpu.sync_copy(dst_hbm.at[pl.ds(b, EK)], idx_d)
                pltpu.async_copy(tab_hbm.at[idx_s], rows, sem).wait()
                pltpu.sync_copy(rows, acc.at[idx_d], add=True)
                return carry
            lax.fori_loop(0, per_s // EK, step, 0)

        zero_own_range()
        plsc.subcore_barrier()

        outs = (o0_hbm, o1_hbm, o2_hbm, o3_hbm)
        for ci in range(NC):
            @pl.when(c == ci)
            def _(ci=ci):
                edges(2 * ci)
                plsc.subcore_barrier()
                dump(outs[2 * ci])
                pltpu.sync_copy(zr_hbm, zbuf)
                zero_own_range()
                plsc.subcore_barrier()
                edges(2 * ci + 1)
                plsc.subcore_barrier()
                dump(outs[2 * ci + 1])

    return agg_kernel(hs_tab, src4, dst, zrows)


_BLK = 1000
_P = jax.lax.Precision.HIGHEST


def _dot(a, b):
    return jnp.dot(a, b, precision=_P, preferred_element_type=jnp.float32)


def _silu(v):
    return v * jax.nn.sigmoid(v)


def _quarters(hn, dinv):
    q = jnp.stack([hn[:, 0:64], hn[:, 64:128],
                   hn[:, 128:192], hn[:, 192:256]], axis=0)
    return q * dinv[None]


def _embed_kernel(x_ref, w_ref, b_ref, dg0_ref, dg1_ref,
                  h_ref, hs_ref, dinv_ref):
    h = _dot(x_ref[...], w_ref[...]) + b_ref[...]
    deg = dg0_ref[...] + dg1_ref[...]
    dinv = lax.rsqrt(deg)
    h_ref[...] = h
    hs_ref[...] = _quarters(h, dinv)
    dinv_ref[...] = dinv


def _tc_embed(x, emb_W, emb_b, dg0, dg1):
    grid = N // _BLK
    return pl.pallas_call(
        _embed_kernel,
        grid=(grid,),
        in_specs=[
            pl.BlockSpec((_BLK, 128), lambda i: (i, 0)),
            pl.BlockSpec((128, D), lambda i: (0, 0)),
            pl.BlockSpec((1, D), lambda i: (0, 0)),
            pl.BlockSpec((_BLK, 1), lambda i: (i, 0)),
            pl.BlockSpec((_BLK, 1), lambda i: (i, 0)),
        ],
        out_specs=[
            pl.BlockSpec((_BLK, D), lambda i: (i, 0)),
            pl.BlockSpec((4, _BLK, DQ), lambda i: (0, i, 0)),
            pl.BlockSpec((_BLK, 1), lambda i: (i, 0)),
        ],
        out_shape=[
            jax.ShapeDtypeStruct((N, D), jnp.float32),
            jax.ShapeDtypeStruct((4, N, DQ), jnp.float32),
            jax.ShapeDtypeStruct((N, 1), jnp.float32),
        ],
    )(x, emb_W, emb_b.reshape(1, D), dg0, dg1)


def _mlp_kernel(h_ref, a0_ref, a1_ref, a2_ref, a3_ref, dinv_ref,
                w1_ref, b1_ref, w2_ref, b2_ref, hn_ref, hs_ref):
    dinv = dinv_ref[...]
    h = h_ref[...]
    xc = jnp.concatenate(
        [h, a0_ref[...] * dinv, a1_ref[...] * dinv,
         a2_ref[...] * dinv, a3_ref[...] * dinv], axis=1)
    a = _silu(_dot(xc, w1_ref[...]) + b1_ref[...])
    hn = h + _dot(a, w2_ref[...]) + b2_ref[...]
    hn_ref[...] = hn
    hs_ref[...] = _quarters(hn, dinv)


def _final_kernel(h_ref, a0_ref, a1_ref, a2_ref, a3_ref, dinv_ref,
                  w1_ref, b1_ref, w2_ref, b2_ref,
                  pw1_ref, pb1_ref, pw2_ref, pb2_ref, out_ref):
    dinv = dinv_ref[...]
    h = h_ref[...]
    xc = jnp.concatenate(
        [h, a0_ref[...] * dinv, a1_ref[...] * dinv,
         a2_ref[...] * dinv, a3_ref[...] * dinv], axis=1)
    a = _silu(_dot(xc, w1_ref[...]) + b1_ref[...])
    hn = h + _dot(a, w2_ref[...]) + b2_ref[...]
    p = _dot(hn, pw1_ref[...]) + pb1_ref[...]
    out_ref[...] = _dot(p, pw2_ref[...]) + pb2_ref[...]


def _row_specs():
    return [
        pl.BlockSpec((_BLK, D), lambda i: (i, 0)),
        pl.BlockSpec((_BLK, DQ), lambda i: (i, 0)),
        pl.BlockSpec((_BLK, DQ), lambda i: (i, 0)),
        pl.BlockSpec((_BLK, DQ), lambda i: (i, 0)),
        pl.BlockSpec((_BLK, DQ), lambda i: (i, 0)),
        pl.BlockSpec((_BLK, 1), lambda i: (i, 0)),
    ]


def _w_specs():
    return [
        pl.BlockSpec((2 * D, D), lambda i: (0, 0)),
        pl.BlockSpec((1, D), lambda i: (0, 0)),
        pl.BlockSpec((D, D), lambda i: (0, 0)),
        pl.BlockSpec((1, D), lambda i: (0, 0)),
    ]


def _tc_mlp(h, aggs, dinv, lp):
    grid = N // _BLK
    return pl.pallas_call(
        _mlp_kernel,
        grid=(grid,),
        in_specs=_row_specs() + _w_specs(),
        out_specs=[
            pl.BlockSpec((_BLK, D), lambda i: (i, 0)),
            pl.BlockSpec((4, _BLK, DQ), lambda i: (0, i, 0)),
        ],
        out_shape=[
            jax.ShapeDtypeStruct((N, D), jnp.float32),
            jax.ShapeDtypeStruct((4, N, DQ), jnp.float32),
        ],
    )(h, *aggs, dinv, lp["nW1"], lp["nb1"].reshape(1, D),
      lp["nW2"], lp["nb2"].reshape(1, D))


def _tc_final(h, aggs, dinv, lp, params):
    grid = N // _BLK
    return pl.pallas_call(
        _final_kernel,
        grid=(grid,),
        in_specs=_row_specs() + _w_specs() + [
            pl.BlockSpec((D, D), lambda i: (0, 0)),
            pl.BlockSpec((1, D), lambda i: (0, 0)),
            pl.BlockSpec((D, 128), lambda i: (0, 0)),
            pl.BlockSpec((1, 128), lambda i: (0, 0)),
        ],
        out_specs=pl.BlockSpec((_BLK, 128), lambda i: (i, 0)),
        out_shape=jax.ShapeDtypeStruct((N, 128), jnp.float32),
    )(h, *aggs, dinv, lp["nW1"], lp["nb1"].reshape(1, D),
      lp["nW2"], lp["nb2"].reshape(1, D),
      params["pW1"], params["pb1"].reshape(1, D),
      params["pW2"], params["pb2"].reshape(1, 128))


def kernel(x, edge_index, edge_attr, batch, params):
    src = edge_index[0]
    dst = edge_index[1]
    src4 = (src[None, :] +
            (jnp.arange(4, dtype=src.dtype) * N)[:, None]).reshape(4 * E)

    d0, d1 = _sc_degree(src)
    h, hs, dinv = _tc_embed(x, params["emb_W"], params["emb_b"],
                            d0[:N, None], d1[:N, None])

    for li, lp in enumerate(params["layers"]):
        aggs = _sc_aggregate(hs.reshape(4 * N, DQ), src4, dst)
        if li < len(params["layers"]) - 1:
            h, hs = _tc_mlp(h, aggs, dinv, lp)
        else:
            out = _tc_final(h, aggs, dinv, lp, params)
    return out


# speedup vs baseline: 9.4059x; 2.6695x over previous
import functools

import jax
import jax.numpy as jnp
from jax import lax
from jax.experimental import pallas as pl
from jax.experimental.pallas import tpu as pltpu
from jax.experimental.pallas import tpu_sc as plsc

N = 10000
E = 320000
D = 256
DQ = 64
NC = 2
NS = 16
EK = 80
NP = 10112

_MESH = dict(core_axis_name="c", subcore_axis_name="s")


def _sc_degree(src2):
    ones = jnp.ones((EK,), jnp.float32)
    zer = jnp.zeros((632,), jnp.float32)

    @functools.partial(
        pl.kernel,
        out_type=[jax.ShapeDtypeStruct((NP,), jnp.float32),
                  jax.ShapeDtypeStruct((NP,), jnp.float32)],
        mesh=plsc.VectorSubcoreMesh(**_MESH),
        scratch_types=[
            pltpu.VMEM((E // (NC * NS * EK), EK), jnp.int32),
            pltpu.VMEM((EK,), jnp.float32),
            pltpu.VMEM((632,), jnp.float32),
            pltpu.VMEM_SHARED((NP,), jnp.float32),
        ],
        compiler_params=pltpu.CompilerParams(use_tc_tiling_on_sc=False),
    )
    def deg_kernel(src_hbm, ones_hbm, zer_hbm, out0_hbm, out1_hbm,
                   idx_v, ones_v, stage_v, acc):
        c = lax.axis_index("c")
        s = lax.axis_index("s")

        pltpu.sync_copy(zer_hbm, stage_v)

        @pl.when(s < NS - 1)
        def _():
            pltpu.sync_copy(stage_v, acc.at[pl.ds(s * 632, 632)])

        @pl.when(s == NS - 1)
        def _():
            pltpu.sync_copy(stage_v.at[pl.ds(0, 520)],
                            acc.at[pl.ds((NS - 1) * 632, 520)])

        pltpu.sync_copy(ones_hbm, ones_v)

        nchunk = E // (NC * NS * EK)
        w = c * NS + s
        pltpu.sync_copy(src_hbm.at[pl.ds(w * nchunk, nchunk)], idx_v)
        plsc.subcore_barrier()

        def step(k, carry):
            pltpu.sync_copy(ones_v, acc.at[idx_v.at[k]], add=True)
            return carry

        lax.fori_loop(0, nchunk, step, 0)
        plsc.subcore_barrier()

        for ci, out_hbm in enumerate((out0_hbm, out1_hbm)):
            @pl.when((c == ci) & (s < NS - 1))
            def _(out_hbm=out_hbm):
                pltpu.sync_copy(acc.at[pl.ds(s * 632, 632)], stage_v)
                pltpu.sync_copy(stage_v, out_hbm.at[pl.ds(s * 632, 632)])

            @pl.when((c == ci) & (s == NS - 1))
            def _(out_hbm=out_hbm):
                pltpu.sync_copy(acc.at[pl.ds((NS - 1) * 632, 520)],
                                stage_v.at[pl.ds(0, 520)])
                pltpu.sync_copy(stage_v.at[pl.ds(0, 520)],
                                out_hbm.at[pl.ds((NS - 1) * 632, 520)])

    return deg_kernel(src2, ones, zer)


_NCH = E // (NS * EK)


def _sc_aggregate(hs_tab, src4, dst2):
    zrows = jnp.zeros((128, DQ), jnp.float32)

    @functools.partial(
        pl.kernel,
        out_type=[jax.ShapeDtypeStruct((N, DQ), jnp.float32)] * 4,
        mesh=plsc.VectorSubcoreMesh(**_MESH),
        scratch_types=[
            pltpu.VMEM((_NCH, EK), jnp.int32),
            pltpu.VMEM((_NCH, EK), jnp.int32),
            pltpu.VMEM((EK, DQ), jnp.float32),
            pltpu.VMEM((EK, DQ), jnp.float32),
            pltpu.VMEM((128, DQ), jnp.float32),
            pltpu.VMEM_SHARED((N, DQ), jnp.float32),
            pltpu.SemaphoreType.DMA,
            pltpu.SemaphoreType.DMA,
        ],
        compiler_params=pltpu.CompilerParams(use_tc_tiling_on_sc=False),
    )
    def agg_kernel(tab_hbm, src_hbm, dst_hbm, zr_hbm,
                   o0_hbm, o1_hbm, o2_hbm, o3_hbm,
                   idx_s, idx_d, rows0, rows1, zbuf, acc, sem0, sem1):
        c = lax.axis_index("c")
        s = lax.axis_index("s")

        pltpu.sync_copy(zr_hbm, zbuf)

        def zero_own_range():
            @pl.when(s < NS - 1)
            def _():
                def zs(j, carry):
                    pltpu.sync_copy(zbuf, acc.at[pl.ds(s * 640 + j * 128, 128)])
                    return carry
                lax.fori_loop(0, 5, zs, 0)

            @pl.when(s == NS - 1)
            def _():
                def zs(j, carry):
                    pltpu.sync_copy(zbuf.at[pl.ds(0, 80)],
                                    acc.at[pl.ds(9600 + j * 80, 80)])
                    return carry
                lax.fori_loop(0, 5, zs, 0)

        def dump(out_hbm):
            @pl.when(s < NS - 1)
            def _():
                def dsx(j, carry):
                    r = s * 640 + j * 128
                    pltpu.sync_copy(acc.at[pl.ds(r, 128)], rows_stage)
                    pltpu.sync_copy(rows_stage, out_hbm.at[pl.ds(r, 128)])
                    return carry
                lax.fori_loop(0, 5, dsx, 0)

            @pl.when(s == NS - 1)
            def _():
                def dsx(j, carry):
                    r = 9600 + j * 80
                    pltpu.sync_copy(acc.at[pl.ds(r, 80)],
                                    rows_stage.at[pl.ds(0, 80)])
                    pltpu.sync_copy(rows_stage.at[pl.ds(0, 80)],
                                    out_hbm.at[pl.ds(r, 80)])
                    return carry
                lax.fori_loop(0, 5, dsx, 0)

        rows_stage = zbuf

        pltpu.sync_copy(dst_hbm.at[pl.ds(s * _NCH, _NCH)], idx_d)

        def edges(qi):
            pltpu.sync_copy(src_hbm.at[qi, pl.ds(s * _NCH, _NCH)], idx_s)

            def gather(k, rows, sem):
                return pltpu.async_copy(tab_hbm.at[idx_s.at[k]], rows, sem)

            def gwait(k, rows, sem):
                pltpu.make_async_copy(tab_hbm.at[idx_s.at[k]], rows, sem).wait()

            def scat(k, rows):
                pltpu.sync_copy(rows, acc.at[idx_d.at[k]], add=True)

            gather(0, rows0, sem0)
            gather(1, rows1, sem1)

            def pair(p, carry):
                k0 = 2 * p
                gwait(k0, rows0, sem0)
                scat(k0, rows0)

                @pl.when(p < _NCH // 2 - 1)
                def _():
                    gather(k0 + 2, rows0, sem0)

                gwait(k0 + 1, rows1, sem1)
                scat(k0 + 1, rows1)

                @pl.when(p < _NCH // 2 - 1)
                def _():
                    gather(k0 + 3, rows1, sem1)

                return carry

            lax.fori_loop(0, _NCH // 2, pair, 0)

        zero_own_range()
        plsc.subcore_barrier()

        outs = (o0_hbm, o1_hbm, o2_hbm, o3_hbm)
        for ci in range(NC):
            @pl.when(c == ci)
            def _(ci=ci):
                edges(2 * ci)
                plsc.subcore_barrier()
                dump(outs[2 * ci])
                pltpu.sync_copy(zr_hbm, zbuf)
                zero_own_range()
                plsc.subcore_barrier()
                edges(2 * ci + 1)
                plsc.subcore_barrier()
                dump(outs[2 * ci + 1])

    return agg_kernel(hs_tab, src4, dst2, zrows)


_BLK = 1000
_P = jax.lax.Precision.HIGHEST


def _dot(a, b):
    return jnp.dot(a, b, precision=_P, preferred_element_type=jnp.float32)


def _silu(v):
    return v * jax.nn.sigmoid(v)


def _quarters(hn, dinv):
    q = jnp.stack([hn[:, 0:64], hn[:, 64:128],
                   hn[:, 128:192], hn[:, 192:256]], axis=0)
    return q * dinv[None]


def _embed_kernel(x_ref, w_ref, b_ref, dg0_ref, dg1_ref,
                  h_ref, hs_ref, dinv_ref):
    h = _dot(x_ref[...], w_ref[...]) + b_ref[...]
    deg = dg0_ref[...] + dg1_ref[...]
    dinv = lax.rsqrt(deg)
    h_ref[...] = h
    hs_ref[...] = _quarters(h, dinv)
    dinv_ref[...] = dinv


def _tc_embed(x, emb_W, emb_b, dg0, dg1):
    grid = N // _BLK
    return pl.pallas_call(
        _embed_kernel,
        grid=(grid,),
        in_specs=[
            pl.BlockSpec((_BLK, 128), lambda i: (i, 0)),
            pl.BlockSpec((128, D), lambda i: (0, 0)),
            pl.BlockSpec((1, D), lambda i: (0, 0)),
            pl.BlockSpec((_BLK, 1), lambda i: (i, 0)),
            pl.BlockSpec((_BLK, 1), lambda i: (i, 0)),
        ],
        out_specs=[
            pl.BlockSpec((_BLK, D), lambda i: (i, 0)),
            pl.BlockSpec((4, _BLK, DQ), lambda i: (0, i, 0)),
            pl.BlockSpec((_BLK, 1), lambda i: (i, 0)),
        ],
        out_shape=[
            jax.ShapeDtypeStruct((N, D), jnp.float32),
            jax.ShapeDtypeStruct((4, N, DQ), jnp.float32),
            jax.ShapeDtypeStruct((N, 1), jnp.float32),
        ],
    )(x, emb_W, emb_b.reshape(1, D), dg0, dg1)


def _mlp_kernel(h_ref, a0_ref, a1_ref, a2_ref, a3_ref, dinv_ref,
                w1_ref, b1_ref, w2_ref, b2_ref, hn_ref, hs_ref):
    dinv = dinv_ref[...]
    h = h_ref[...]
    xc = jnp.concatenate(
        [h, a0_ref[...] * dinv, a1_ref[...] * dinv,
         a2_ref[...] * dinv, a3_ref[...] * dinv], axis=1)
    a = _silu(_dot(xc, w1_ref[...]) + b1_ref[...])
    hn = h + _dot(a, w2_ref[...]) + b2_ref[...]
    hn_ref[...] = hn
    hs_ref[...] = _quarters(hn, dinv)


def _final_kernel(h_ref, a0_ref, a1_ref, a2_ref, a3_ref, dinv_ref,
                  w1_ref, b1_ref, w2_ref, b2_ref,
                  pw1_ref, pb1_ref, pw2_ref, pb2_ref, out_ref):
    dinv = dinv_ref[...]
    h = h_ref[...]
    xc = jnp.concatenate(
        [h, a0_ref[...] * dinv, a1_ref[...] * dinv,
         a2_ref[...] * dinv, a3_ref[...] * dinv], axis=1)
    a = _silu(_dot(xc, w1_ref[...]) + b1_ref[...])
    hn = h + _dot(a, w2_ref[...]) + b2_ref[...]
    p = _dot(hn, pw1_ref[...]) + pb1_ref[...]
    out_ref[...] = _dot(p, pw2_ref[...]) + pb2_ref[...]


def _row_specs():
    return [
        pl.BlockSpec((_BLK, D), lambda i: (i, 0)),
        pl.BlockSpec((_BLK, DQ), lambda i: (i, 0)),
        pl.BlockSpec((_BLK, DQ), lambda i: (i, 0)),
        pl.BlockSpec((_BLK, DQ), lambda i: (i, 0)),
        pl.BlockSpec((_BLK, DQ), lambda i: (i, 0)),
        pl.BlockSpec((_BLK, 1), lambda i: (i, 0)),
    ]


def _w_specs():
    return [
        pl.BlockSpec((2 * D, D), lambda i: (0, 0)),
        pl.BlockSpec((1, D), lambda i: (0, 0)),
        pl.BlockSpec((D, D), lambda i: (0, 0)),
        pl.BlockSpec((1, D), lambda i: (0, 0)),
    ]


def _tc_mlp(h, aggs, dinv, lp):
    grid = N // _BLK
    return pl.pallas_call(
        _mlp_kernel,
        grid=(grid,),
        in_specs=_row_specs() + _w_specs(),
        out_specs=[
            pl.BlockSpec((_BLK, D), lambda i: (i, 0)),
            pl.BlockSpec((4, _BLK, DQ), lambda i: (0, i, 0)),
        ],
        out_shape=[
            jax.ShapeDtypeStruct((N, D), jnp.float32),
            jax.ShapeDtypeStruct((4, N, DQ), jnp.float32),
        ],
    )(h, *aggs, dinv, lp["nW1"], lp["nb1"].reshape(1, D),
      lp["nW2"], lp["nb2"].reshape(1, D))


def _tc_final(h, aggs, dinv, lp, params):
    grid = N // _BLK
    return pl.pallas_call(
        _final_kernel,
        grid=(grid,),
        in_specs=_row_specs() + _w_specs() + [
            pl.BlockSpec((D, D), lambda i: (0, 0)),
            pl.BlockSpec((1, D), lambda i: (0, 0)),
            pl.BlockSpec((D, 128), lambda i: (0, 0)),
            pl.BlockSpec((1, 128), lambda i: (0, 0)),
        ],
        out_specs=pl.BlockSpec((_BLK, 128), lambda i: (i, 0)),
        out_shape=jax.ShapeDtypeStruct((N, 128), jnp.float32),
    )(h, *aggs, dinv, lp["nW1"], lp["nb1"].reshape(1, D),
      lp["nW2"], lp["nb2"].reshape(1, D),
      params["pW1"], params["pb1"].reshape(1, D),
      params["pW2"], params["pb2"].reshape(1, 128))


def kernel(x, edge_index, edge_attr, batch, params):
    src = edge_index[0]
    dst = edge_index[1]
    src4 = (src[None, :] +
            (jnp.arange(4, dtype=src.dtype) * N)[:, None]
            ).reshape(4, E // EK, EK)
    src2 = src.reshape(E // EK, EK)
    dst2 = dst.reshape(E // EK, EK)

    d0, d1 = _sc_degree(src2)
    h, hs, dinv = _tc_embed(x, params["emb_W"], params["emb_b"],
                            d0[:N, None], d1[:N, None])

    for li, lp in enumerate(params["layers"]):
        aggs = _sc_aggregate(hs.reshape(4 * N, DQ), src4, dst2)
        if li < len(params["layers"]) - 1:
            h, hs = _tc_mlp(h, aggs, dinv, lp)
        else:
            out = _tc_final(h, aggs, dinv, lp, params)
    return out


# speedup vs baseline: 10.3140x; 1.0965x over previous
import functools

import jax
import jax.numpy as jnp
from jax import lax
from jax.experimental import pallas as pl
from jax.experimental.pallas import tpu as pltpu
from jax.experimental.pallas import tpu_sc as plsc

N = 10000
E = 320000
D = 256
DQ = 64
NC = 2
NS = 16
EK = 80
NP = 10112

_MESH = dict(core_axis_name="c", subcore_axis_name="s")


def _sc_degree(src2):
    ones = jnp.ones((EK,), jnp.float32)
    zer = jnp.zeros((632,), jnp.float32)

    @functools.partial(
        pl.kernel,
        out_type=[jax.ShapeDtypeStruct((NP,), jnp.float32),
                  jax.ShapeDtypeStruct((NP,), jnp.float32)],
        mesh=plsc.VectorSubcoreMesh(**_MESH),
        scratch_types=[
            pltpu.VMEM((E // (NC * NS * EK), EK), jnp.int32),
            pltpu.VMEM((EK,), jnp.float32),
            pltpu.VMEM((632,), jnp.float32),
            pltpu.VMEM_SHARED((NP,), jnp.float32),
        ],
        compiler_params=pltpu.CompilerParams(use_tc_tiling_on_sc=False),
    )
    def deg_kernel(src_hbm, ones_hbm, zer_hbm, out0_hbm, out1_hbm,
                   idx_v, ones_v, stage_v, acc):
        c = lax.axis_index("c")
        s = lax.axis_index("s")

        pltpu.sync_copy(zer_hbm, stage_v)

        @pl.when(s < NS - 1)
        def _():
            pltpu.sync_copy(stage_v, acc.at[pl.ds(s * 632, 632)])

        @pl.when(s == NS - 1)
        def _():
            pltpu.sync_copy(stage_v.at[pl.ds(0, 520)],
                            acc.at[pl.ds((NS - 1) * 632, 520)])

        pltpu.sync_copy(ones_hbm, ones_v)

        nchunk = E // (NC * NS * EK)
        w = c * NS + s
        pltpu.sync_copy(src_hbm.at[pl.ds(w * nchunk, nchunk)], idx_v)
        plsc.subcore_barrier()

        def step(k, carry):
            pltpu.sync_copy(ones_v, acc.at[idx_v.at[k]], add=True)
            return carry

        lax.fori_loop(0, nchunk, step, 0)
        plsc.subcore_barrier()

        for ci, out_hbm in enumerate((out0_hbm, out1_hbm)):
            @pl.when((c == ci) & (s < NS - 1))
            def _(out_hbm=out_hbm):
                pltpu.sync_copy(acc.at[pl.ds(s * 632, 632)], stage_v)
                pltpu.sync_copy(stage_v, out_hbm.at[pl.ds(s * 632, 632)])

            @pl.when((c == ci) & (s == NS - 1))
            def _(out_hbm=out_hbm):
                pltpu.sync_copy(acc.at[pl.ds((NS - 1) * 632, 520)],
                                stage_v.at[pl.ds(0, 520)])
                pltpu.sync_copy(stage_v.at[pl.ds(0, 520)],
                                out_hbm.at[pl.ds((NS - 1) * 632, 520)])

    return deg_kernel(src2, ones, zer)


_NCH = E // (NS * EK)


def _sc_aggregate(hs_tab, src4, dst2):
    zrows = jnp.zeros((128, DQ), jnp.float32)

    @functools.partial(
        pl.kernel,
        out_type=[jax.ShapeDtypeStruct((N, DQ), jnp.float32)] * 4,
        mesh=plsc.VectorSubcoreMesh(**_MESH),
        scratch_types=[
            pltpu.VMEM((_NCH, EK), jnp.int32),
            pltpu.VMEM((_NCH, EK), jnp.int32),
            pltpu.VMEM((EK, DQ), jnp.float32),
            pltpu.VMEM((EK, DQ), jnp.float32),
            pltpu.VMEM((EK, DQ), jnp.float32),
            pltpu.VMEM((EK, DQ), jnp.float32),
            pltpu.VMEM((128, DQ), jnp.float32),
            pltpu.VMEM_SHARED((N, DQ), jnp.float32),
            [pltpu.SemaphoreType.DMA] * 4,
            [pltpu.SemaphoreType.DMA] * 4,
        ],
        compiler_params=pltpu.CompilerParams(use_tc_tiling_on_sc=False),
    )
    def agg_kernel(tab_hbm, src_hbm, dst_hbm, zr_hbm,
                   o0_hbm, o1_hbm, o2_hbm, o3_hbm,
                   idx_s, idx_d, rows0, rows1, rows2, rows3,
                   zbuf, acc, gsems, ssems):
        c = lax.axis_index("c")
        s = lax.axis_index("s")

        pltpu.sync_copy(zr_hbm, zbuf)

        def zero_own_range():
            @pl.when(s < NS - 1)
            def _():
                def zs(j, carry):
                    pltpu.sync_copy(zbuf, acc.at[pl.ds(s * 640 + j * 128, 128)])
                    return carry
                lax.fori_loop(0, 5, zs, 0)

            @pl.when(s == NS - 1)
            def _():
                def zs(j, carry):
                    pltpu.sync_copy(zbuf.at[pl.ds(0, 80)],
                                    acc.at[pl.ds(9600 + j * 80, 80)])
                    return carry
                lax.fori_loop(0, 5, zs, 0)

        def dump(out_hbm):
            @pl.when(s < NS - 1)
            def _():
                def dsx(j, carry):
                    r = s * 640 + j * 128
                    pltpu.sync_copy(acc.at[pl.ds(r, 128)], rows_stage)
                    pltpu.sync_copy(rows_stage, out_hbm.at[pl.ds(r, 128)])
                    return carry
                lax.fori_loop(0, 5, dsx, 0)

            @pl.when(s == NS - 1)
            def _():
                def dsx(j, carry):
                    r = 9600 + j * 80
                    pltpu.sync_copy(acc.at[pl.ds(r, 80)],
                                    rows_stage.at[pl.ds(0, 80)])
                    pltpu.sync_copy(rows_stage.at[pl.ds(0, 80)],
                                    out_hbm.at[pl.ds(r, 80)])
                    return carry
                lax.fori_loop(0, 5, dsx, 0)

        rows_stage = zbuf

        pltpu.sync_copy(dst_hbm.at[pl.ds(s * _NCH, _NCH)], idx_d)

        def edges(qi):
            pltpu.sync_copy(src_hbm.at[qi, pl.ds(s * _NCH, _NCH)], idx_s)
            bufs = (rows0, rows1, rows2, rows3)

            def gissue(k, b):
                pltpu.async_copy(tab_hbm.at[idx_s.at[k]], bufs[b], gsems[b])

            def gwait(k, b):
                pltpu.make_async_copy(
                    tab_hbm.at[idx_s.at[k]], bufs[b], gsems[b]).wait()

            def sissue(k, b):
                pltpu.async_copy(bufs[b], acc.at[idx_d.at[k]], ssems[b],
                                 add=True)

            def swait(k, b):
                pltpu.make_async_copy(
                    bufs[b], acc.at[idx_d.at[k]], ssems[b]).wait()

            gissue(0, 0)
            gissue(1, 1)
            gwait(0, 0)
            sissue(0, 0)
            gissue(2, 2)
            gwait(1, 1)
            sissue(1, 1)
            gissue(3, 3)

            def round4(m, carry):
                for j in range(4):
                    k = 2 + 4 * m + j
                    b = (2 + j) % 4
                    swait(k - 2, j)
                    gwait(k, b)
                    sissue(k, b)

                    @pl.when(k + 2 < _NCH)
                    def _(k=k, j=j):
                        gissue(k + 2, j)
                return carry

            lax.fori_loop(0, (_NCH - 2) // 4, round4, 0)
            swait(_NCH - 2, (_NCH - 2) % 4)
            swait(_NCH - 1, (_NCH - 1) % 4)

        zero_own_range()
        plsc.subcore_barrier()

        outs = (o0_hbm, o1_hbm, o2_hbm, o3_hbm)
        for ci in range(NC):
            @pl.when(c == ci)
            def _(ci=ci):
                edges(2 * ci)
                plsc.subcore_barrier()
                dump(outs[2 * ci])
                pltpu.sync_copy(zr_hbm, zbuf)
                zero_own_range()
                plsc.subcore_barrier()
                edges(2 * ci + 1)
                plsc.subcore_barrier()
                dump(outs[2 * ci + 1])

    return agg_kernel(hs_tab, src4, dst2, zrows)


_BLK = 1000
_P = jax.lax.Precision.HIGHEST


def _dot(a, b):
    return jnp.dot(a, b, precision=_P, preferred_element_type=jnp.float32)


def _silu(v):
    return v * jax.nn.sigmoid(v)


def _quarters(hn, dinv):
    q = jnp.stack([hn[:, 0:64], hn[:, 64:128],
                   hn[:, 128:192], hn[:, 192:256]], axis=0)
    return q * dinv[None]


def _embed_kernel(x_ref, w_ref, b_ref, dg0_ref, dg1_ref,
                  h_ref, hs_ref, dinv_ref):
    h = _dot(x_ref[...], w_ref[...]) + b_ref[...]
    deg = dg0_ref[...] + dg1_ref[...]
    dinv = lax.rsqrt(deg)
    h_ref[...] = h
    hs_ref[...] = _quarters(h, dinv)
    dinv_ref[...] = dinv


def _tc_embed(x, emb_W, emb_b, dg0, dg1):
    grid = N // _BLK
    return pl.pallas_call(
        _embed_kernel,
        grid=(grid,),
        in_specs=[
            pl.BlockSpec((_BLK, 128), lambda i: (i, 0)),
            pl.BlockSpec((128, D), lambda i: (0, 0)),
            pl.BlockSpec((1, D), lambda i: (0, 0)),
            pl.BlockSpec((_BLK, 1), lambda i: (i, 0)),
            pl.BlockSpec((_BLK, 1), lambda i: (i, 0)),
        ],
        out_specs=[
            pl.BlockSpec((_BLK, D), lambda i: (i, 0)),
            pl.BlockSpec((4, _BLK, DQ), lambda i: (0, i, 0)),
            pl.BlockSpec((_BLK, 1), lambda i: (i, 0)),
        ],
        out_shape=[
            jax.ShapeDtypeStruct((N, D), jnp.float32),
            jax.ShapeDtypeStruct((4, N, DQ), jnp.float32),
            jax.ShapeDtypeStruct((N, 1), jnp.float32),
        ],
    )(x, emb_W, emb_b.reshape(1, D), dg0, dg1)


def _mlp_kernel(h_ref, a0_ref, a1_ref, a2_ref, a3_ref, dinv_ref,
                w1_ref, b1_ref, w2_ref, b2_ref, hn_ref, hs_ref):
    dinv = dinv_ref[...]
    h = h_ref[...]
    xc = jnp.concatenate(
        [h, a0_ref[...] * dinv, a1_ref[...] * dinv,
         a2_ref[...] * dinv, a3_ref[...] * dinv], axis=1)
    a = _silu(_dot(xc, w1_ref[...]) + b1_ref[...])
    hn = h + _dot(a, w2_ref[...]) + b2_ref[...]
    hn_ref[...] = hn
    hs_ref[...] = _quarters(hn, dinv)


def _final_kernel(h_ref, a0_ref, a1_ref, a2_ref, a3_ref, dinv_ref,
                  w1_ref, b1_ref, w2_ref, b2_ref,
                  pw1_ref, pb1_ref, pw2_ref, pb2_ref, out_ref):
    dinv = dinv_ref[...]
    h = h_ref[...]
    xc = jnp.concatenate(
        [h, a0_ref[...] * dinv, a1_ref[...] * dinv,
         a2_ref[...] * dinv, a3_ref[...] * dinv], axis=1)
    a = _silu(_dot(xc, w1_ref[...]) + b1_ref[...])
    hn = h + _dot(a, w2_ref[...]) + b2_ref[...]
    p = _dot(hn, pw1_ref[...]) + pb1_ref[...]
    out_ref[...] = _dot(p, pw2_ref[...]) + pb2_ref[...]


def _row_specs():
    return [
        pl.BlockSpec((_BLK, D), lambda i: (i, 0)),
        pl.BlockSpec((_BLK, DQ), lambda i: (i, 0)),
        pl.BlockSpec((_BLK, DQ), lambda i: (i, 0)),
        pl.BlockSpec((_BLK, DQ), lambda i: (i, 0)),
        pl.BlockSpec((_BLK, DQ), lambda i: (i, 0)),
        pl.BlockSpec((_BLK, 1), lambda i: (i, 0)),
    ]


def _w_specs():
    return [
        pl.BlockSpec((2 * D, D), lambda i: (0, 0)),
        pl.BlockSpec((1, D), lambda i: (0, 0)),
        pl.BlockSpec((D, D), lambda i: (0, 0)),
        pl.BlockSpec((1, D), lambda i: (0, 0)),
    ]


def _tc_mlp(h, aggs, dinv, lp):
    grid = N // _BLK
    return pl.pallas_call(
        _mlp_kernel,
        grid=(grid,),
        in_specs=_row_specs() + _w_specs(),
        out_specs=[
            pl.BlockSpec((_BLK, D), lambda i: (i, 0)),
            pl.BlockSpec((4, _BLK, DQ), lambda i: (0, i, 0)),
        ],
        out_shape=[
            jax.ShapeDtypeStruct((N, D), jnp.float32),
            jax.ShapeDtypeStruct((4, N, DQ), jnp.float32),
        ],
    )(h, *aggs, dinv, lp["nW1"], lp["nb1"].reshape(1, D),
      lp["nW2"], lp["nb2"].reshape(1, D))


def _tc_final(h, aggs, dinv, lp, params):
    grid = N // _BLK
    return pl.pallas_call(
        _final_kernel,
        grid=(grid,),
        in_specs=_row_specs() + _w_specs() + [
            pl.BlockSpec((D, D), lambda i: (0, 0)),
            pl.BlockSpec((1, D), lambda i: (0, 0)),
            pl.BlockSpec((D, 128), lambda i: (0, 0)),
            pl.BlockSpec((1, 128), lambda i: (0, 0)),
        ],
        out_specs=pl.BlockSpec((_BLK, 128), lambda i: (i, 0)),
        out_shape=jax.ShapeDtypeStruct((N, 128), jnp.float32),
    )(h, *aggs, dinv, lp["nW1"], lp["nb1"].reshape(1, D),
      lp["nW2"], lp["nb2"].reshape(1, D),
      params["pW1"], params["pb1"].reshape(1, D),
      params["pW2"], params["pb2"].reshape(1, 128))


def kernel(x, edge_index, edge_attr, batch, params):
    src = edge_index[0]
    dst = edge_index[1]
    src4 = (src[None, :] +
            (jnp.arange(4, dtype=src.dtype) * N)[:, None]
            ).reshape(4, E // EK, EK)
    src2 = src.reshape(E // EK, EK)
    dst2 = dst.reshape(E // EK, EK)

    d0, d1 = _sc_degree(src2)
    h, hs, dinv = _tc_embed(x, params["emb_W"], params["emb_b"],
                            d0[:N, None], d1[:N, None])

    for li, lp in enumerate(params["layers"]):
        aggs = _sc_aggregate(hs.reshape(4 * N, DQ), src4, dst2)
        if li < len(params["layers"]) - 1:
            h, hs = _tc_mlp(h, aggs, dinv, lp)
        else:
            out = _tc_final(h, aggs, dinv, lp, params)
    return out


# speedup vs baseline: 11.3306x; 1.0986x over previous
import functools

import jax
import jax.numpy as jnp
from jax import lax
from jax.experimental import pallas as pl
from jax.experimental.pallas import tpu as pltpu
from jax.experimental.pallas import tpu_sc as plsc

N = 10000
E = 320000
D = 256
DQ = 64
NC = 2
NS = 16
EK = 80
NP = 10112

_MESH = dict(core_axis_name="c", subcore_axis_name="s")


def _sc_degree(src2):
    ones = jnp.ones((EK,), jnp.float32)
    zer = jnp.zeros((632,), jnp.float32)

    @functools.partial(
        pl.kernel,
        out_type=[jax.ShapeDtypeStruct((NP,), jnp.float32),
                  jax.ShapeDtypeStruct((NP,), jnp.float32)],
        mesh=plsc.VectorSubcoreMesh(**_MESH),
        scratch_types=[
            pltpu.VMEM((E // (NC * NS * EK), EK), jnp.int32),
            pltpu.VMEM((EK,), jnp.float32),
            pltpu.VMEM((632,), jnp.float32),
            pltpu.VMEM_SHARED((NP,), jnp.float32),
        ],
        compiler_params=pltpu.CompilerParams(use_tc_tiling_on_sc=False),
    )
    def deg_kernel(src_hbm, ones_hbm, zer_hbm, out0_hbm, out1_hbm,
                   idx_v, ones_v, stage_v, acc):
        c = lax.axis_index("c")
        s = lax.axis_index("s")

        pltpu.sync_copy(zer_hbm, stage_v)

        @pl.when(s < NS - 1)
        def _():
            pltpu.sync_copy(stage_v, acc.at[pl.ds(s * 632, 632)])

        @pl.when(s == NS - 1)
        def _():
            pltpu.sync_copy(stage_v.at[pl.ds(0, 520)],
                            acc.at[pl.ds((NS - 1) * 632, 520)])

        pltpu.sync_copy(ones_hbm, ones_v)

        nchunk = E // (NC * NS * EK)
        w = c * NS + s
        pltpu.sync_copy(src_hbm.at[pl.ds(w * nchunk, nchunk)], idx_v)
        plsc.subcore_barrier()

        def step(k, carry):
            pltpu.sync_copy(ones_v, acc.at[idx_v.at[k]], add=True)
            return carry

        lax.fori_loop(0, nchunk, step, 0)
        plsc.subcore_barrier()

        for ci, out_hbm in enumerate((out0_hbm, out1_hbm)):
            @pl.when((c == ci) & (s < NS - 1))
            def _(out_hbm=out_hbm):
                pltpu.sync_copy(acc.at[pl.ds(s * 632, 632)], stage_v)
                pltpu.sync_copy(stage_v, out_hbm.at[pl.ds(s * 632, 632)])

            @pl.when((c == ci) & (s == NS - 1))
            def _(out_hbm=out_hbm):
                pltpu.sync_copy(acc.at[pl.ds((NS - 1) * 632, 520)],
                                stage_v.at[pl.ds(0, 520)])
                pltpu.sync_copy(stage_v.at[pl.ds(0, 520)],
                                out_hbm.at[pl.ds((NS - 1) * 632, 520)])

    return deg_kernel(src2, ones, zer)


EKA = 128
EP = 321536
_NCH = EP // (NS * EKA)
NA = N + 8


def _sc_aggregate(hs_tab, src4, dst2):
    zrows = jnp.zeros((128, DQ), jnp.float32)

    @functools.partial(
        pl.kernel,
        out_type=[jax.ShapeDtypeStruct((N, DQ), jnp.float32)] * 4,
        mesh=plsc.VectorSubcoreMesh(**_MESH),
        scratch_types=[
            pltpu.VMEM((_NCH, EKA), jnp.int32),
            pltpu.VMEM((_NCH, EKA), jnp.int32),
            pltpu.VMEM((EKA, DQ), jnp.float32),
            pltpu.VMEM((EKA, DQ), jnp.float32),
            pltpu.VMEM((EKA, DQ), jnp.float32),
            pltpu.VMEM((EKA, DQ), jnp.float32),
            pltpu.VMEM((128, DQ), jnp.float32),
            pltpu.VMEM_SHARED((NA, DQ), jnp.float32),
            [pltpu.SemaphoreType.DMA] * 4,
            [pltpu.SemaphoreType.DMA] * 4,
        ],
        compiler_params=pltpu.CompilerParams(use_tc_tiling_on_sc=False),
    )
    def agg_kernel(tab_hbm, src_hbm, dst_hbm, zr_hbm,
                   o0_hbm, o1_hbm, o2_hbm, o3_hbm,
                   idx_s, idx_d, rows0, rows1, rows2, rows3,
                   zbuf, acc, gsems, ssems):
        c = lax.axis_index("c")
        s = lax.axis_index("s")

        pltpu.sync_copy(zr_hbm, zbuf)

        def zero_own_range():
            @pl.when(s < NS - 1)
            def _():
                def zs(j, carry):
                    pltpu.sync_copy(zbuf, acc.at[pl.ds(s * 640 + j * 128, 128)])
                    return carry
                lax.fori_loop(0, 5, zs, 0)

            @pl.when(s == NS - 1)
            def _():
                def zs(j, carry):
                    pltpu.sync_copy(zbuf.at[pl.ds(0, 80)],
                                    acc.at[pl.ds(9600 + j * 80, 80)])
                    return carry
                lax.fori_loop(0, 5, zs, 0)

        def dump(out_hbm):
            @pl.when(s < NS - 1)
            def _():
                def dsx(j, carry):
                    r = s * 640 + j * 128
                    pltpu.sync_copy(acc.at[pl.ds(r, 128)], rows_stage)
                    pltpu.sync_copy(rows_stage, out_hbm.at[pl.ds(r, 128)])
                    return carry
                lax.fori_loop(0, 5, dsx, 0)

            @pl.when(s == NS - 1)
            def _():
                def dsx(j, carry):
                    r = 9600 + j * 80
                    pltpu.sync_copy(acc.at[pl.ds(r, 80)],
                                    rows_stage.at[pl.ds(0, 80)])
                    pltpu.sync_copy(rows_stage.at[pl.ds(0, 80)],
                                    out_hbm.at[pl.ds(r, 80)])
                    return carry
                lax.fori_loop(0, 5, dsx, 0)

        rows_stage = zbuf

        pltpu.sync_copy(dst_hbm.at[pl.ds(s * _NCH, _NCH)], idx_d)

        def edges(qi):
            pltpu.sync_copy(src_hbm.at[qi, pl.ds(s * _NCH, _NCH)], idx_s)
            bufs = (rows0, rows1, rows2, rows3)

            def gissue(k, b):
                pltpu.async_copy(tab_hbm.at[idx_s.at[k]], bufs[b], gsems[b])

            def gwait(k, b):
                pltpu.make_async_copy(
                    tab_hbm.at[idx_s.at[k]], bufs[b], gsems[b]).wait()

            def sissue(k, b):
                pltpu.async_copy(bufs[b], acc.at[idx_d.at[k]], ssems[b],
                                 add=True)

            def swait(k, b):
                pltpu.make_async_copy(
                    bufs[b], acc.at[idx_d.at[k]], ssems[b]).wait()

            gissue(0, 0)
            gissue(1, 1)
            gwait(0, 0)
            sissue(0, 0)
            gissue(2, 2)
            gwait(1, 1)
            sissue(1, 1)
            gissue(3, 3)

            def round4(m, carry):
                for j in range(4):
                    k = 2 + 4 * m + j
                    b = (2 + j) % 4
                    swait(k - 2, j)
                    gwait(k, b)
                    sissue(k, b)

                    @pl.when(k + 2 < _NCH)
                    def _(k=k, j=j):
                        gissue(k + 2, j)
                return carry

            steady = (_NCH - 2) // 4
            lax.fori_loop(0, steady, round4, 0)
            for k in range(2 + 4 * steady, _NCH):
                swait(k - 2, (k - 2) % 4)
                gwait(k, k % 4)
                sissue(k, k % 4)
                if k + 2 < _NCH:
                    gissue(k + 2, (k + 2) % 4)
            swait(_NCH - 2, (_NCH - 2) % 4)
            swait(_NCH - 1, (_NCH - 1) % 4)

        zero_own_range()
        plsc.subcore_barrier()

        outs = (o0_hbm, o1_hbm, o2_hbm, o3_hbm)
        for ci in range(NC):
            @pl.when(c == ci)
            def _(ci=ci):
                edges(2 * ci)
                plsc.subcore_barrier()
                dump(outs[2 * ci])
                pltpu.sync_copy(zr_hbm, zbuf)
                zero_own_range()
                plsc.subcore_barrier()
                edges(2 * ci + 1)
                plsc.subcore_barrier()
                dump(outs[2 * ci + 1])

    return agg_kernel(hs_tab, src4, dst2, zrows)


_BLK = 1000
_P = jax.lax.Precision.HIGHEST


def _dot(a, b):
    return jnp.dot(a, b, precision=_P, preferred_element_type=jnp.float32)


def _silu(v):
    return v * jax.nn.sigmoid(v)


def _quarters(hn, dinv):
    q = jnp.stack([hn[:, 0:64], hn[:, 64:128],
                   hn[:, 128:192], hn[:, 192:256]], axis=0)
    return q * dinv[None]


def _embed_kernel(x_ref, w_ref, b_ref, dg0_ref, dg1_ref,
                  h_ref, hs_ref, dinv_ref):
    h = _dot(x_ref[...], w_ref[...]) + b_ref[...]
    deg = dg0_ref[...] + dg1_ref[...]
    dinv = lax.rsqrt(deg)
    h_ref[...] = h
    hs_ref[...] = _quarters(h, dinv)
    dinv_ref[...] = dinv


def _tc_embed(x, emb_W, emb_b, dg0, dg1):
    grid = N // _BLK
    return pl.pallas_call(
        _embed_kernel,
        grid=(grid,),
        in_specs=[
            pl.BlockSpec((_BLK, 128), lambda i: (i, 0)),
            pl.BlockSpec((128, D), lambda i: (0, 0)),
            pl.BlockSpec((1, D), lambda i: (0, 0)),
            pl.BlockSpec((_BLK, 1), lambda i: (i, 0)),
            pl.BlockSpec((_BLK, 1), lambda i: (i, 0)),
        ],
        out_specs=[
            pl.BlockSpec((_BLK, D), lambda i: (i, 0)),
            pl.BlockSpec((4, _BLK, DQ), lambda i: (0, i, 0)),
            pl.BlockSpec((_BLK, 1), lambda i: (i, 0)),
        ],
        out_shape=[
            jax.ShapeDtypeStruct((N, D), jnp.float32),
            jax.ShapeDtypeStruct((4, N, DQ), jnp.float32),
            jax.ShapeDtypeStruct((N, 1), jnp.float32),
        ],
    )(x, emb_W, emb_b.reshape(1, D), dg0, dg1)


def _mlp_kernel(h_ref, a0_ref, a1_ref, a2_ref, a3_ref, dinv_ref,
                w1_ref, b1_ref, w2_ref, b2_ref, hn_ref, hs_ref):
    dinv = dinv_ref[...]
    h = h_ref[...]
    xc = jnp.concatenate(
        [h, a0_ref[...] * dinv, a1_ref[...] * dinv,
         a2_ref[...] * dinv, a3_ref[...] * dinv], axis=1)
    a = _silu(_dot(xc, w1_ref[...]) + b1_ref[...])
    hn = h + _dot(a, w2_ref[...]) + b2_ref[...]
    hn_ref[...] = hn
    hs_ref[...] = _quarters(hn, dinv)


def _final_kernel(h_ref, a0_ref, a1_ref, a2_ref, a3_ref, dinv_ref,
                  w1_ref, b1_ref, w2_ref, b2_ref,
                  pw1_ref, pb1_ref, pw2_ref, pb2_ref, out_ref):
    dinv = dinv_ref[...]
    h = h_ref[...]
    xc = jnp.concatenate(
        [h, a0_ref[...] * dinv, a1_ref[...] * dinv,
         a2_ref[...] * dinv, a3_ref[...] * dinv], axis=1)
    a = _silu(_dot(xc, w1_ref[...]) + b1_ref[...])
    hn = h + _dot(a, w2_ref[...]) + b2_ref[...]
    p = _dot(hn, pw1_ref[...]) + pb1_ref[...]
    out_ref[...] = _dot(p, pw2_ref[...]) + pb2_ref[...]


def _row_specs():
    return [
        pl.BlockSpec((_BLK, D), lambda i: (i, 0)),
        pl.BlockSpec((_BLK, DQ), lambda i: (i, 0)),
        pl.BlockSpec((_BLK, DQ), lambda i: (i, 0)),
        pl.BlockSpec((_BLK, DQ), lambda i: (i, 0)),
        pl.BlockSpec((_BLK, DQ), lambda i: (i, 0)),
        pl.BlockSpec((_BLK, 1), lambda i: (i, 0)),
    ]


def _w_specs():
    return [
        pl.BlockSpec((2 * D, D), lambda i: (0, 0)),
        pl.BlockSpec((1, D), lambda i: (0, 0)),
        pl.BlockSpec((D, D), lambda i: (0, 0)),
        pl.BlockSpec((1, D), lambda i: (0, 0)),
    ]


def _tc_mlp(h, aggs, dinv, lp):
    grid = N // _BLK
    return pl.pallas_call(
        _mlp_kernel,
        grid=(grid,),
        in_specs=_row_specs() + _w_specs(),
        out_specs=[
            pl.BlockSpec((_BLK, D), lambda i: (i, 0)),
            pl.BlockSpec((4, _BLK, DQ), lambda i: (0, i, 0)),
        ],
        out_shape=[
            jax.ShapeDtypeStruct((N, D), jnp.float32),
            jax.ShapeDtypeStruct((4, N, DQ), jnp.float32),
        ],
    )(h, *aggs, dinv, lp["nW1"], lp["nb1"].reshape(1, D),
      lp["nW2"], lp["nb2"].reshape(1, D))


def _tc_final(h, aggs, dinv, lp, params):
    grid = N // _BLK
    return pl.pallas_call(
        _final_kernel,
        grid=(grid,),
        in_specs=_row_specs() + _w_specs() + [
            pl.BlockSpec((D, D), lambda i: (0, 0)),
            pl.BlockSpec((1, D), lambda i: (0, 0)),
            pl.BlockSpec((D, 128), lambda i: (0, 0)),
            pl.BlockSpec((1, 128), lambda i: (0, 0)),
        ],
        out_specs=pl.BlockSpec((_BLK, 128), lambda i: (i, 0)),
        out_shape=jax.ShapeDtypeStruct((N, 128), jnp.float32),
    )(h, *aggs, dinv, lp["nW1"], lp["nb1"].reshape(1, D),
      lp["nW2"], lp["nb2"].reshape(1, D),
      params["pW1"], params["pb1"].reshape(1, D),
      params["pW2"], params["pb2"].reshape(1, 128))


def kernel(x, edge_index, edge_attr, batch, params):
    src = edge_index[0]
    dst = edge_index[1]
    pad = jnp.arange(EP - E, dtype=src.dtype)
    src_p = jnp.concatenate([src, pad % N])
    dst_p = jnp.concatenate([dst, N + (pad % 8)])
    src4 = (src_p[None, :] +
            (jnp.arange(4, dtype=src.dtype) * N)[:, None]
            ).reshape(4, EP // EKA, EKA)
    src2 = src.reshape(E // EK, EK)
    dst2 = dst_p.reshape(EP // EKA, EKA)

    d0, d1 = _sc_degree(src2)
    h, hs, dinv = _tc_embed(x, params["emb_W"], params["emb_b"],
                            d0[:N, None], d1[:N, None])

    for li, lp in enumerate(params["layers"]):
        aggs = _sc_aggregate(hs.reshape(4 * N, DQ), src4, dst2)
        if li < len(params["layers"]) - 1:
            h, hs = _tc_mlp(h, aggs, dinv, lp)
        else:
            out = _tc_final(h, aggs, dinv, lp, params)
    return out


# speedup vs baseline: 12.6276x; 1.1145x over previous
import functools

import jax
import jax.numpy as jnp
from jax import lax
from jax.experimental import pallas as pl
from jax.experimental.pallas import tpu as pltpu
from jax.experimental.pallas import tpu_sc as plsc

N = 10000
E = 320000
D = 256
DQ = 64
NC = 2
NS = 16
EK = 80
NP = 10112

_MESH = dict(core_axis_name="c", subcore_axis_name="s")


def _sc_degree(src2):
    ones = jnp.ones((EK,), jnp.float32)
    zer = jnp.zeros((632,), jnp.float32)

    @functools.partial(
        pl.kernel,
        out_type=[jax.ShapeDtypeStruct((NP,), jnp.float32),
                  jax.ShapeDtypeStruct((NP,), jnp.float32)],
        mesh=plsc.VectorSubcoreMesh(**_MESH),
        scratch_types=[
            pltpu.VMEM((E // (NC * NS * EK), EK), jnp.int32),
            pltpu.VMEM((EK,), jnp.float32),
            pltpu.VMEM((632,), jnp.float32),
            pltpu.VMEM_SHARED((NP,), jnp.float32),
        ],
        compiler_params=pltpu.CompilerParams(use_tc_tiling_on_sc=False),
    )
    def deg_kernel(src_hbm, ones_hbm, zer_hbm, out0_hbm, out1_hbm,
                   idx_v, ones_v, stage_v, acc):
        c = lax.axis_index("c")
        s = lax.axis_index("s")

        pltpu.sync_copy(zer_hbm, stage_v)

        @pl.when(s < NS - 1)
        def _():
            pltpu.sync_copy(stage_v, acc.at[pl.ds(s * 632, 632)])

        @pl.when(s == NS - 1)
        def _():
            pltpu.sync_copy(stage_v.at[pl.ds(0, 520)],
                            acc.at[pl.ds((NS - 1) * 632, 520)])

        pltpu.sync_copy(ones_hbm, ones_v)

        nchunk = E // (NC * NS * EK)
        w = c * NS + s
        pltpu.sync_copy(src_hbm.at[pl.ds(w * nchunk, nchunk)], idx_v)
        plsc.subcore_barrier()

        def step(k, carry):
            pltpu.sync_copy(ones_v, acc.at[idx_v.at[k]], add=True)
            return carry

        lax.fori_loop(0, nchunk, step, 0)
        plsc.subcore_barrier()

        for ci, out_hbm in enumerate((out0_hbm, out1_hbm)):
            @pl.when((c == ci) & (s < NS - 1))
            def _(out_hbm=out_hbm):
                pltpu.sync_copy(acc.at[pl.ds(s * 632, 632)], stage_v)
                pltpu.sync_copy(stage_v, out_hbm.at[pl.ds(s * 632, 632)])

            @pl.when((c == ci) & (s == NS - 1))
            def _(out_hbm=out_hbm):
                pltpu.sync_copy(acc.at[pl.ds((NS - 1) * 632, 520)],
                                stage_v.at[pl.ds(0, 520)])
                pltpu.sync_copy(stage_v.at[pl.ds(0, 520)],
                                out_hbm.at[pl.ds((NS - 1) * 632, 520)])

    return deg_kernel(src2, ones, zer)


EKA = 128
EP = 321536
_NCH = EP // (NS * EKA)
NA = N + 8


def _sc_aggregate(hs_tab, src2, dst2):
    zrows = jnp.zeros((128, DQ), jnp.float32)

    @functools.partial(
        pl.kernel,
        out_type=[jax.ShapeDtypeStruct((N, DQ), jnp.float32)] * 4,
        mesh=plsc.VectorSubcoreMesh(**_MESH),
        scratch_types=[
            pltpu.VMEM((_NCH, EKA), jnp.int32),
            pltpu.VMEM((_NCH, EKA), jnp.int32),
            pltpu.VMEM((EKA, DQ), jnp.float32),
            pltpu.VMEM((EKA, DQ), jnp.float32),
            pltpu.VMEM((EKA, DQ), jnp.float32),
            pltpu.VMEM((EKA, DQ), jnp.float32),
            pltpu.VMEM((128, DQ), jnp.float32),
            pltpu.VMEM_SHARED((NA, DQ), jnp.float32),
            [pltpu.SemaphoreType.DMA] * 4,
            [pltpu.SemaphoreType.DMA] * 4,
        ],
        compiler_params=pltpu.CompilerParams(use_tc_tiling_on_sc=False),
    )
    def agg_kernel(tab_hbm, src_hbm, dst_hbm, zr_hbm,
                   o0_hbm, o1_hbm, o2_hbm, o3_hbm,
                   idx_s, idx_d, rows0, rows1, rows2, rows3,
                   zbuf, acc, gsems, ssems):
        c = lax.axis_index("c")
        s = lax.axis_index("s")

        pltpu.sync_copy(zr_hbm, zbuf)

        def zero_own_range():
            @pl.when(s < NS - 1)
            def _():
                def zs(j, carry):
                    pltpu.sync_copy(zbuf, acc.at[pl.ds(s * 640 + j * 128, 128)])
                    return carry
                lax.fori_loop(0, 5, zs, 0)

            @pl.when(s == NS - 1)
            def _():
                def zs(j, carry):
                    pltpu.sync_copy(zbuf.at[pl.ds(0, 80)],
                                    acc.at[pl.ds(9600 + j * 80, 80)])
                    return carry
                lax.fori_loop(0, 5, zs, 0)

        def dump(out_hbm):
            @pl.when(s < NS - 1)
            def _():
                def dsx(j, carry):
                    r = s * 640 + j * 128
                    pltpu.sync_copy(acc.at[pl.ds(r, 128)], rows_stage)
                    pltpu.sync_copy(rows_stage, out_hbm.at[pl.ds(r, 128)])
                    return carry
                lax.fori_loop(0, 5, dsx, 0)

            @pl.when(s == NS - 1)
            def _():
                def dsx(j, carry):
                    r = 9600 + j * 80
                    pltpu.sync_copy(acc.at[pl.ds(r, 80)],
                                    rows_stage.at[pl.ds(0, 80)])
                    pltpu.sync_copy(rows_stage.at[pl.ds(0, 80)],
                                    out_hbm.at[pl.ds(r, 80)])
                    return carry
                lax.fori_loop(0, 5, dsx, 0)

        rows_stage = zbuf

        pltpu.sync_copy(dst_hbm.at[pl.ds(s * _NCH, _NCH)], idx_d)
        pltpu.sync_copy(src_hbm.at[pl.ds(s * _NCH, _NCH)], idx_s)

        def edges(qi):
            tab_q = tab_hbm.at[pl.ds(qi * N, N)]
            bufs = (rows0, rows1, rows2, rows3)

            def gissue(k, b):
                pltpu.async_copy(tab_q.at[idx_s.at[k]], bufs[b], gsems[b])

            def gwait(k, b):
                pltpu.make_async_copy(
                    tab_q.at[idx_s.at[k]], bufs[b], gsems[b]).wait()

            def sissue(k, b):
                pltpu.async_copy(bufs[b], acc.at[idx_d.at[k]], ssems[b],
                                 add=True)

            def swait(k, b):
                pltpu.make_async_copy(
                    bufs[b], acc.at[idx_d.at[k]], ssems[b]).wait()

            gissue(0, 0)
            gissue(1, 1)
            gwait(0, 0)
            sissue(0, 0)
            gissue(2, 2)
            gwait(1, 1)
            sissue(1, 1)
            gissue(3, 3)

            def round4(m, carry):
                for j in range(4):
                    k = 2 + 4 * m + j
                    b = (2 + j) % 4
                    swait(k - 2, j)
                    gwait(k, b)
                    sissue(k, b)

                    @pl.when(k + 2 < _NCH)
                    def _(k=k, j=j):
                        gissue(k + 2, j)
                return carry

            steady = (_NCH - 2) // 4
            lax.fori_loop(0, steady, round4, 0)
            for k in range(2 + 4 * steady, _NCH):
                swait(k - 2, (k - 2) % 4)
                gwait(k, k % 4)
                sissue(k, k % 4)
                if k + 2 < _NCH:
                    gissue(k + 2, (k + 2) % 4)
            swait(_NCH - 2, (_NCH - 2) % 4)
            swait(_NCH - 1, (_NCH - 1) % 4)

        zero_own_range()
        plsc.subcore_barrier()

        outs = (o0_hbm, o1_hbm, o2_hbm, o3_hbm)
        for ci in range(NC):
            @pl.when(c == ci)
            def _(ci=ci):
                edges(2 * ci)
                plsc.subcore_barrier()
                dump(outs[2 * ci])
                pltpu.sync_copy(zr_hbm, zbuf)
                zero_own_range()
                plsc.subcore_barrier()
                edges(2 * ci + 1)
                plsc.subcore_barrier()
                dump(outs[2 * ci + 1])

    return agg_kernel(hs_tab, src2, dst2, zrows)


_BLK = 1000
_P = jax.lax.Precision.DEFAULT


def _dot(a, b):
    return jnp.dot(a, b, precision=_P, preferred_element_type=jnp.float32)


def _silu(v):
    return v * jax.nn.sigmoid(v)


def _quarters(hn, dinv):
    q = jnp.stack([hn[:, 0:64], hn[:, 64:128],
                   hn[:, 128:192], hn[:, 192:256]], axis=0)
    return q * dinv[None]


def _embed_kernel(x_ref, w_ref, b_ref, dg0_ref, dg1_ref,
                  h_ref, hs_ref, dinv_ref):
    h = _dot(x_ref[...], w_ref[...]) + b_ref[...]
    deg = dg0_ref[...] + dg1_ref[...]
    dinv = lax.rsqrt(deg)
    h_ref[...] = h
    hs_ref[...] = _quarters(h, dinv)
    dinv_ref[...] = dinv


def _tc_embed(x, emb_W, emb_b, dg0, dg1):
    grid = N // _BLK
    return pl.pallas_call(
        _embed_kernel,
        grid=(grid,),
        in_specs=[
            pl.BlockSpec((_BLK, 128), lambda i: (i, 0)),
            pl.BlockSpec((128, D), lambda i: (0, 0)),
            pl.BlockSpec((1, D), lambda i: (0, 0)),
            pl.BlockSpec((_BLK, 1), lambda i: (i, 0)),
            pl.BlockSpec((_BLK, 1), lambda i: (i, 0)),
        ],
        out_specs=[
            pl.BlockSpec((_BLK, D), lambda i: (i, 0)),
            pl.BlockSpec((4, _BLK, DQ), lambda i: (0, i, 0)),
            pl.BlockSpec((_BLK, 1), lambda i: (i, 0)),
        ],
        out_shape=[
            jax.ShapeDtypeStruct((N, D), jnp.float32),
            jax.ShapeDtypeStruct((4, N, DQ), jnp.float32),
            jax.ShapeDtypeStruct((N, 1), jnp.float32),
        ],
    )(x, emb_W, emb_b.reshape(1, D), dg0, dg1)


def _mlp_kernel(h_ref, a0_ref, a1_ref, a2_ref, a3_ref, dinv_ref,
                w1_ref, b1_ref, w2_ref, b2_ref, hn_ref, hs_ref):
    dinv = dinv_ref[...]
    h = h_ref[...]
    xc = jnp.concatenate(
        [h, a0_ref[...] * dinv, a1_ref[...] * dinv,
         a2_ref[...] * dinv, a3_ref[...] * dinv], axis=1)
    a = _silu(_dot(xc, w1_ref[...]) + b1_ref[...])
    hn = h + _dot(a, w2_ref[...]) + b2_ref[...]
    hn_ref[...] = hn
    hs_ref[...] = _quarters(hn, dinv)


def _final_kernel(h_ref, a0_ref, a1_ref, a2_ref, a3_ref, dinv_ref,
                  w1_ref, b1_ref, w2_ref, b2_ref,
                  pw1_ref, pb1_ref, pw2_ref, pb2_ref, out_ref):
    dinv = dinv_ref[...]
    h = h_ref[...]
    xc = jnp.concatenate(
        [h, a0_ref[...] * dinv, a1_ref[...] * dinv,
         a2_ref[...] * dinv, a3_ref[...] * dinv], axis=1)
    a = _silu(_dot(xc, w1_ref[...]) + b1_ref[...])
    hn = h + _dot(a, w2_ref[...]) + b2_ref[...]
    p = _dot(hn, pw1_ref[...]) + pb1_ref[...]
    out_ref[...] = _dot(p, pw2_ref[...]) + pb2_ref[...]


def _row_specs():
    return [
        pl.BlockSpec((_BLK, D), lambda i: (i, 0)),
        pl.BlockSpec((_BLK, DQ), lambda i: (i, 0)),
        pl.BlockSpec((_BLK, DQ), lambda i: (i, 0)),
        pl.BlockSpec((_BLK, DQ), lambda i: (i, 0)),
        pl.BlockSpec((_BLK, DQ), lambda i: (i, 0)),
        pl.BlockSpec((_BLK, 1), lambda i: (i, 0)),
    ]


def _w_specs():
    return [
        pl.BlockSpec((2 * D, D), lambda i: (0, 0)),
        pl.BlockSpec((1, D), lambda i: (0, 0)),
        pl.BlockSpec((D, D), lambda i: (0, 0)),
        pl.BlockSpec((1, D), lambda i: (0, 0)),
    ]


def _tc_mlp(h, aggs, dinv, lp):
    grid = N // _BLK
    return pl.pallas_call(
        _mlp_kernel,
        grid=(grid,),
        in_specs=_row_specs() + _w_specs(),
        out_specs=[
            pl.BlockSpec((_BLK, D), lambda i: (i, 0)),
            pl.BlockSpec((4, _BLK, DQ), lambda i: (0, i, 0)),
        ],
        out_shape=[
            jax.ShapeDtypeStruct((N, D), jnp.float32),
            jax.ShapeDtypeStruct((4, N, DQ), jnp.float32),
        ],
    )(h, *aggs, dinv, lp["nW1"], lp["nb1"].reshape(1, D),
      lp["nW2"], lp["nb2"].reshape(1, D))


def _tc_final(h, aggs, dinv, lp, params):
    grid = N // _BLK
    return pl.pallas_call(
        _final_kernel,
        grid=(grid,),
        in_specs=_row_specs() + _w_specs() + [
            pl.BlockSpec((D, D), lambda i: (0, 0)),
            pl.BlockSpec((1, D), lambda i: (0, 0)),
            pl.BlockSpec((D, 128), lambda i: (0, 0)),
            pl.BlockSpec((1, 128), lambda i: (0, 0)),
        ],
        out_specs=pl.BlockSpec((_BLK, 128), lambda i: (i, 0)),
        out_shape=jax.ShapeDtypeStruct((N, 128), jnp.float32),
    )(h, *aggs, dinv, lp["nW1"], lp["nb1"].reshape(1, D),
      lp["nW2"], lp["nb2"].reshape(1, D),
      params["pW1"], params["pb1"].reshape(1, D),
      params["pW2"], params["pb2"].reshape(1, 128))


def kernel(x, edge_index, edge_attr, batch, params):
    src = edge_index[0]
    dst = edge_index[1]
    pad = jnp.arange(EP - E, dtype=src.dtype)
    src_pa = jnp.concatenate([src, pad % N]).reshape(EP // EKA, EKA)
    dst_pa = jnp.concatenate([dst, N + (pad % 8)]).reshape(EP // EKA, EKA)
    src2 = src.reshape(E // EK, EK)

    d0, d1 = _sc_degree(src2)
    h, hs, dinv = _tc_embed(x, params["emb_W"], params["emb_b"],
                            d0[:N, None], d1[:N, None])

    for li, lp in enumerate(params["layers"]):
        aggs = _sc_aggregate(hs.reshape(4 * N, DQ), src_pa, dst_pa)
        if li < len(params["layers"]) - 1:
            h, hs = _tc_mlp(h, aggs, dinv, lp)
        else:
            out = _tc_final(h, aggs, dinv, lp, params)
    return out


# speedup vs baseline: 14.4601x; 1.1451x over previous
import functools

import jax
import jax.numpy as jnp
from jax import lax
from jax.experimental import pallas as pl
from jax.experimental.pallas import tpu as pltpu
from jax.experimental.pallas import tpu_sc as plsc

N = 10000
E = 320000
D = 256
DQ = 64
NC = 2
NS = 16
EK = 80
NP = 10112

_MESH = dict(core_axis_name="c", subcore_axis_name="s")


def _sc_degree(src2):
    ones = jnp.ones((EK,), jnp.float32)
    zer = jnp.zeros((632,), jnp.float32)

    @functools.partial(
        pl.kernel,
        out_type=[jax.ShapeDtypeStruct((NP,), jnp.float32),
                  jax.ShapeDtypeStruct((NP,), jnp.float32)],
        mesh=plsc.VectorSubcoreMesh(**_MESH),
        scratch_types=[
            pltpu.VMEM((E // (NC * NS * EK), EK), jnp.int32),
            pltpu.VMEM((EK,), jnp.float32),
            pltpu.VMEM((632,), jnp.float32),
            pltpu.VMEM_SHARED((NP,), jnp.float32),
        ],
        compiler_params=pltpu.CompilerParams(use_tc_tiling_on_sc=False),
    )
    def deg_kernel(src_hbm, ones_hbm, zer_hbm, out0_hbm, out1_hbm,
                   idx_v, ones_v, stage_v, acc):
        c = lax.axis_index("c")
        s = lax.axis_index("s")

        pltpu.sync_copy(zer_hbm, stage_v)

        @pl.when(s < NS - 1)
        def _():
            pltpu.sync_copy(stage_v, acc.at[pl.ds(s * 632, 632)])

        @pl.when(s == NS - 1)
        def _():
            pltpu.sync_copy(stage_v.at[pl.ds(0, 520)],
                            acc.at[pl.ds((NS - 1) * 632, 520)])

        pltpu.sync_copy(ones_hbm, ones_v)

        nchunk = E // (NC * NS * EK)
        w = c * NS + s
        pltpu.sync_copy(src_hbm.at[pl.ds(w * nchunk, nchunk)], idx_v)
        plsc.subcore_barrier()

        def step(k, carry):
            pltpu.sync_copy(ones_v, acc.at[idx_v.at[k]], add=True)
            return carry

        lax.fori_loop(0, nchunk, step, 0)
        plsc.subcore_barrier()

        for ci, out_hbm in enumerate((out0_hbm, out1_hbm)):
            @pl.when((c == ci) & (s < NS - 1))
            def _(out_hbm=out_hbm):
                pltpu.sync_copy(acc.at[pl.ds(s * 632, 632)], stage_v)
                pltpu.sync_copy(stage_v, out_hbm.at[pl.ds(s * 632, 632)])

            @pl.when((c == ci) & (s == NS - 1))
            def _(out_hbm=out_hbm):
                pltpu.sync_copy(acc.at[pl.ds((NS - 1) * 632, 520)],
                                stage_v.at[pl.ds(0, 520)])
                pltpu.sync_copy(stage_v.at[pl.ds(0, 520)],
                                out_hbm.at[pl.ds((NS - 1) * 632, 520)])

    return deg_kernel(src2, ones, zer)


EKA = 128
EP = 321536
_NCH = EP // (NS * EKA)
NA = N + 8
_NB = 5
_AH = 3


def _sc_aggregate(hs_tab, src2, dst2):
    zrows = jnp.zeros((128, DQ), jnp.float32)

    @functools.partial(
        pl.kernel,
        out_type=[jax.ShapeDtypeStruct((N, DQ), jnp.float32)] * 4,
        mesh=plsc.VectorSubcoreMesh(**_MESH),
        scratch_types=[
            pltpu.VMEM((_NCH, EKA), jnp.int32),
            pltpu.VMEM((_NCH, EKA), jnp.int32),
            [pltpu.VMEM((EKA, DQ), jnp.float32)] * _NB,
            pltpu.VMEM((128, DQ), jnp.float32),
            pltpu.VMEM_SHARED((NA, DQ), jnp.float32),
            [pltpu.SemaphoreType.DMA] * _NB,
            [pltpu.SemaphoreType.DMA] * _NB,
        ],
        compiler_params=pltpu.CompilerParams(use_tc_tiling_on_sc=False),
    )
    def agg_kernel(tab_hbm, src_hbm, dst_hbm, zr_hbm,
                   o0_hbm, o1_hbm, o2_hbm, o3_hbm,
                   idx_s, idx_d, bufs, zbuf, acc, gsems, ssems):
        c = lax.axis_index("c")
        s = lax.axis_index("s")

        pltpu.sync_copy(zr_hbm, zbuf)

        def zero_own_range():
            @pl.when(s < NS - 1)
            def _():
                def zs(j, carry):
                    pltpu.sync_copy(zbuf, acc.at[pl.ds(s * 640 + j * 128, 128)])
                    return carry
                lax.fori_loop(0, 5, zs, 0)

            @pl.when(s == NS - 1)
            def _():
                def zs(j, carry):
                    pltpu.sync_copy(zbuf.at[pl.ds(0, 80)],
                                    acc.at[pl.ds(9600 + j * 80, 80)])
                    return carry
                lax.fori_loop(0, 5, zs, 0)

        def dump(out_hbm):
            @pl.when(s < NS - 1)
            def _():
                def dsx(j, carry):
                    r = s * 640 + j * 128
                    pltpu.sync_copy(acc.at[pl.ds(r, 128)], rows_stage)
                    pltpu.sync_copy(rows_stage, out_hbm.at[pl.ds(r, 128)])
                    return carry
                lax.fori_loop(0, 5, dsx, 0)

            @pl.when(s == NS - 1)
            def _():
                def dsx(j, carry):
                    r = 9600 + j * 80
                    pltpu.sync_copy(acc.at[pl.ds(r, 80)],
                                    rows_stage.at[pl.ds(0, 80)])
                    pltpu.sync_copy(rows_stage.at[pl.ds(0, 80)],
                                    out_hbm.at[pl.ds(r, 80)])
                    return carry
                lax.fori_loop(0, 5, dsx, 0)

        rows_stage = zbuf

        pltpu.sync_copy(dst_hbm.at[pl.ds(s * _NCH, _NCH)], idx_d)
        pltpu.sync_copy(src_hbm.at[pl.ds(s * _NCH, _NCH)], idx_s)

        def edges(qi):
            tab_q = tab_hbm.at[pl.ds(qi * N, N)]

            def gissue(k, b):
                pltpu.async_copy(tab_q.at[idx_s.at[k]], bufs[b], gsems[b])

            def gwait(k, b):
                pltpu.make_async_copy(
                    tab_q.at[idx_s.at[k]], bufs[b], gsems[b]).wait()

            def sissue(k, b):
                pltpu.async_copy(bufs[b], acc.at[idx_d.at[k]], ssems[b],
                                 add=True)

            def swait(k, b):
                pltpu.make_async_copy(
                    bufs[b], acc.at[idx_d.at[k]], ssems[b]).wait()

            def step(k, jj, guard):
                gwait(k, jj)
                sissue(k, jj)
                kw = k - (_NB - _AH)
                ka = k + _AH
                bw = (jj - (_NB - _AH)) % _NB
                if guard:
                    @pl.when(kw >= 0)
                    def _():
                        swait(kw, bw)

                    @pl.when(ka < _NCH)
                    def _():
                        gissue(ka, bw)
                else:
                    if kw >= 0:
                        swait(kw, bw)
                    if ka < _NCH:
                        gissue(ka, bw)

            for k in range(_AH):
                gissue(k, k)

            def round_nb(m, carry):
                for j in range(_NB):
                    step(_NB * m + j, j, guard=True)
                return carry

            steady = _NCH // _NB
            lax.fori_loop(0, steady, round_nb, 0)
            for k in range(_NB * steady, _NCH):
                step(k, k % _NB, guard=False)
            for k in range(_NCH - (_NB - _AH), _NCH):
                swait(k, k % _NB)

        zero_own_range()
        plsc.subcore_barrier()

        outs = (o0_hbm, o1_hbm, o2_hbm, o3_hbm)
        for ci in range(NC):
            @pl.when(c == ci)
            def _(ci=ci):
                edges(2 * ci)
                plsc.subcore_barrier()
                dump(outs[2 * ci])
                pltpu.sync_copy(zr_hbm, zbuf)
                zero_own_range()
                plsc.subcore_barrier()
                edges(2 * ci + 1)
                plsc.subcore_barrier()
                dump(outs[2 * ci + 1])

    return agg_kernel(hs_tab, src2, dst2, zrows)


_BLK = 1000
_P = jax.lax.Precision.DEFAULT


def _dot(a, b):
    return jnp.dot(a, b, precision=_P, preferred_element_type=jnp.float32)


def _silu(v):
    return v * jax.nn.sigmoid(v)


def _quarters(hn, dinv):
    q = jnp.stack([hn[:, 0:64], hn[:, 64:128],
                   hn[:, 128:192], hn[:, 192:256]], axis=0)
    return q * dinv[None]


def _embed_kernel(x_ref, w_ref, b_ref, dg0_ref, dg1_ref,
                  h_ref, hs_ref, dinv_ref):
    h = _dot(x_ref[...], w_ref[...]) + b_ref[...]
    deg = dg0_ref[...] + dg1_ref[...]
    dinv = lax.rsqrt(deg)
    h_ref[...] = h
    hs_ref[...] = _quarters(h, dinv)
    dinv_ref[...] = dinv


def _tc_embed(x, emb_W, emb_b, dg0, dg1):
    grid = N // _BLK
    return pl.pallas_call(
        _embed_kernel,
        grid=(grid,),
        in_specs=[
            pl.BlockSpec((_BLK, 128), lambda i: (i, 0)),
            pl.BlockSpec((128, D), lambda i: (0, 0)),
            pl.BlockSpec((1, D), lambda i: (0, 0)),
            pl.BlockSpec((_BLK, 1), lambda i: (i, 0)),
            pl.BlockSpec((_BLK, 1), lambda i: (i, 0)),
        ],
        out_specs=[
            pl.BlockSpec((_BLK, D), lambda i: (i, 0)),
            pl.BlockSpec((4, _BLK, DQ), lambda i: (0, i, 0)),
            pl.BlockSpec((_BLK, 1), lambda i: (i, 0)),
        ],
        out_shape=[
            jax.ShapeDtypeStruct((N, D), jnp.float32),
            jax.ShapeDtypeStruct((4, N, DQ), jnp.float32),
            jax.ShapeDtypeStruct((N, 1), jnp.float32),
        ],
    )(x, emb_W, emb_b.reshape(1, D), dg0, dg1)


def _mlp_kernel(h_ref, a0_ref, a1_ref, a2_ref, a3_ref, dinv_ref,
                w1_ref, b1_ref, w2_ref, b2_ref, hn_ref, hs_ref):
    dinv = dinv_ref[...]
    h = h_ref[...]
    xc = jnp.concatenate(
        [h, a0_ref[...] * dinv, a1_ref[...] * dinv,
         a2_ref[...] * dinv, a3_ref[...] * dinv], axis=1)
    a = _silu(_dot(xc, w1_ref[...]) + b1_ref[...])
    hn = h + _dot(a, w2_ref[...]) + b2_ref[...]
    hn_ref[...] = hn
    hs_ref[...] = _quarters(hn, dinv)


def _final_kernel(h_ref, a0_ref, a1_ref, a2_ref, a3_ref, dinv_ref,
                  w1_ref, b1_ref, w2_ref, b2_ref,
                  pw1_ref, pb1_ref, pw2_ref, pb2_ref, out_ref):
    dinv = dinv_ref[...]
    h = h_ref[...]
    xc = jnp.concatenate(
        [h, a0_ref[...] * dinv, a1_ref[...] * dinv,
         a2_ref[...] * dinv, a3_ref[...] * dinv], axis=1)
    a = _silu(_dot(xc, w1_ref[...]) + b1_ref[...])
    hn = h + _dot(a, w2_ref[...]) + b2_ref[...]
    p = _dot(hn, pw1_ref[...]) + pb1_ref[...]
    out_ref[...] = _dot(p, pw2_ref[...]) + pb2_ref[...]


def _row_specs():
    return [
        pl.BlockSpec((_BLK, D), lambda i: (i, 0)),
        pl.BlockSpec((_BLK, DQ), lambda i: (i, 0)),
        pl.BlockSpec((_BLK, DQ), lambda i: (i, 0)),
        pl.BlockSpec((_BLK, DQ), lambda i: (i, 0)),
        pl.BlockSpec((_BLK, DQ), lambda i: (i, 0)),
        pl.BlockSpec((_BLK, 1), lambda i: (i, 0)),
    ]


def _w_specs():
    return [
        pl.BlockSpec((2 * D, D), lambda i: (0, 0)),
        pl.BlockSpec((1, D), lambda i: (0, 0)),
        pl.BlockSpec((D, D), lambda i: (0, 0)),
        pl.BlockSpec((1, D), lambda i: (0, 0)),
    ]


def _tc_mlp(h, aggs, dinv, lp):
    grid = N // _BLK
    return pl.pallas_call(
        _mlp_kernel,
        grid=(grid,),
        in_specs=_row_specs() + _w_specs(),
        out_specs=[
            pl.BlockSpec((_BLK, D), lambda i: (i, 0)),
            pl.BlockSpec((4, _BLK, DQ), lambda i: (0, i, 0)),
        ],
        out_shape=[
            jax.ShapeDtypeStruct((N, D), jnp.float32),
            jax.ShapeDtypeStruct((4, N, DQ), jnp.float32),
        ],
    )(h, *aggs, dinv, lp["nW1"], lp["nb1"].reshape(1, D),
      lp["nW2"], lp["nb2"].reshape(1, D))


def _tc_final(h, aggs, dinv, lp, params):
    grid = N // _BLK
    return pl.pallas_call(
        _final_kernel,
        grid=(grid,),
        in_specs=_row_specs() + _w_specs() + [
            pl.BlockSpec((D, D), lambda i: (0, 0)),
            pl.BlockSpec((1, D), lambda i: (0, 0)),
            pl.BlockSpec((D, 128), lambda i: (0, 0)),
            pl.BlockSpec((1, 128), lambda i: (0, 0)),
        ],
        out_specs=pl.BlockSpec((_BLK, 128), lambda i: (i, 0)),
        out_shape=jax.ShapeDtypeStruct((N, 128), jnp.float32),
    )(h, *aggs, dinv, lp["nW1"], lp["nb1"].reshape(1, D),
      lp["nW2"], lp["nb2"].reshape(1, D),
      params["pW1"], params["pb1"].reshape(1, D),
      params["pW2"], params["pb2"].reshape(1, 128))


def kernel(x, edge_index, edge_attr, batch, params):
    src = edge_index[0]
    dst = edge_index[1]
    pad = jnp.arange(EP - E, dtype=src.dtype)
    src_pa = jnp.concatenate([src, pad % N]).reshape(EP // EKA, EKA)
    dst_pa = jnp.concatenate([dst, N + (pad % 8)]).reshape(EP // EKA, EKA)
    src2 = src.reshape(E // EK, EK)

    d0, d1 = _sc_degree(src2)
    h, hs, dinv = _tc_embed(x, params["emb_W"], params["emb_b"],
                            d0[:N, None], d1[:N, None])

    for li, lp in enumerate(params["layers"]):
        aggs = _sc_aggregate(hs.reshape(4 * N, DQ), src_pa, dst_pa)
        if li < len(params["layers"]) - 1:
            h, hs = _tc_mlp(h, aggs, dinv, lp)
        else:
            out = _tc_final(h, aggs, dinv, lp, params)
    return out
